# baseline (speedup 1.0000x reference)
import functools

import jax
import jax.numpy as jnp
from jax import lax
from jax.experimental import pallas as pl
from jax.experimental.pallas import tpu as pltpu
from jax.experimental.pallas import tpu_sc as plsc

N_NODES = 10000
N_EDGES = 320000
D = 128

NC = 2
NS = 16
NW = NC * NS
EPW = N_EDGES // NW
K = 80
NCHUNK = EPW // K
TPR = 624
ZR = 16
NZB = TPR // ZR
REM0 = NS * TPR
REM = N_NODES - REM0


def _mm_bias_body(x_ref, w_ref, b_ref, o_ref):
    o_ref[...] = (
        jnp.dot(x_ref[...], w_ref[...], preferred_element_type=jnp.float32)
        + b_ref[...]
    )


def _edge_mm_body(ea_ref, w_ref, o_ref):
    o_ref[...] = jnp.dot(ea_ref[...], w_ref[0],
                         preferred_element_type=jnp.float32)


def _make_update_body():
    def body(h_ref, a0_ref, a1_ref, w_ref, b_ref, g_ref, bb_ref, o_ref):
        s = h_ref[...] + a0_ref[0] + a1_ref[0]
        t = (
            jnp.dot(s, w_ref[0], preferred_element_type=jnp.float32)
            + b_ref[0]
        )
        t = jnp.maximum(t, 0.0)
        mu = jnp.mean(t, axis=-1, keepdims=True)
        var = jnp.mean((t - mu) ** 2, axis=-1, keepdims=True)
        o_ref[...] = (t - mu) * lax.rsqrt(var + 1e-5) * g_ref[0] + bb_ref[0]

    return body


def _make_sc_layer():
    mesh = plsc.VectorSubcoreMesh(core_axis_name="c", subcore_axis_name="s")

    @functools.partial(
        pl.kernel,
        out_type=jax.ShapeDtypeStruct((NC, N_NODES, D), jnp.float32),
        mesh=mesh,
        scratch_types=[
            pltpu.VMEM((4, K), jnp.int32),
            pltpu.VMEM((4, K), jnp.int32),
            pltpu.VMEM((4, K, D), jnp.float32),
            pltpu.VMEM((ZR, D), jnp.float32),
            pltpu.VMEM_SHARED((N_NODES, D), jnp.float32),
            pltpu.SemaphoreType.DMA((4,)),
            pltpu.SemaphoreType.DMA((4,)),
            pltpu.SemaphoreType.DMA((4,)),
            pltpu.SemaphoreType.DMA((4,)),
        ],
    )
    def sc_layer(h_hbm, e_hbm, src_hbm, dst_hbm, out_hbm,
                 sring, dring, ebuf, zbuf, agg_sh, isem, gsem, esem, ssem):
        c = lax.axis_index("c")
        s = lax.axis_index("s")
        w = c * NS + s
        base = w * EPW
        my_row0 = s * TPR

        zero = jnp.zeros((16,), jnp.float32)

        def zrow(r, carry):
            for j in range(8):
                zbuf[r, pl.ds(j * 16, 16)] = zero
            return carry

        lax.fori_loop(0, ZR, zrow, 0)

        def zcopy(i, carry):
            pltpu.sync_copy(zbuf, agg_sh.at[pl.ds(my_row0 + i * ZR, ZR)])
            return carry

        lax.fori_loop(0, NZB, zcopy, 0)

        @pl.when(s == 0)
        def _zero_rem():
            pltpu.sync_copy(zbuf.at[pl.ds(0, REM)],
                            agg_sh.at[pl.ds(REM0, REM)])

        def load_idx(i, sl):
            pltpu.async_copy(src_hbm.at[pl.ds(base + i * K, K)],
                             sring.at[sl], isem.at[sl])
            pltpu.async_copy(dst_hbm.at[pl.ds(base + i * K, K)],
                             dring.at[sl], isem.at[sl])

        def wait_idx(i, sl):
            pltpu.make_async_copy(src_hbm.at[pl.ds(base + i * K, K)],
                                  sring.at[sl], isem.at[sl]).wait()
            pltpu.make_async_copy(dst_hbm.at[pl.ds(base + i * K, K)],
                                  dring.at[sl], isem.at[sl]).wait()

        def load_e(i, b):
            pltpu.async_copy(e_hbm.at[pl.ds(base + i * K, K)],
                             ebuf.at[b], esem.at[b])

        def wait_e(i, b):
            pltpu.make_async_copy(e_hbm.at[pl.ds(base + i * K, K)],
                                  ebuf.at[b], esem.at[b]).wait()

        def issue_gadd(sl, b):
            pltpu.async_copy(h_hbm.at[sring.at[sl]], ebuf.at[b], gsem.at[b],
                             add=True)

        def wait_gadd(sl, b):
            pltpu.make_async_copy(h_hbm.at[sring.at[sl]], ebuf.at[b],
                                  gsem.at[b]).wait()

        def compute(b):
            def crow(r, inner):
                for j in range(8):
                    sl = pl.ds(j * 16, 16)
                    ebuf[b, r, sl] = jnp.maximum(ebuf[b, r, sl], 0.0)
                return inner

            lax.fori_loop(0, K, crow, 0)

        def issue_scatter(b, sl):
            pltpu.async_copy(ebuf.at[b], agg_sh.at[dring.at[sl]],
                             ssem.at[b], add=True)

        def wait_scatter(b, sl):
            pltpu.make_async_copy(ebuf.at[b], agg_sh.at[dring.at[sl]],
                                  ssem.at[b]).wait()

        def step(i, u, last=NCHUNK):
            if isinstance(i, int):
                do_prep = i + 1 < last
                do_wsc = i >= 1
                do_load = i + 3 < last
            else:
                do_prep = do_wsc = do_load = True
            if do_prep:
                wait_idx(i + 1, (u + 1) % 4)
                wait_e(i + 1, (u + 1) % 4)
                issue_gadd((u + 1) % 4, (u + 1) % 4)
            wait_gadd(u % 4, u)
            compute(u)
            issue_scatter(u, u)
            if do_wsc:
                wait_scatter((u + 3) % 4, (u + 3) % 4)
            if do_load:
                load_idx(i + 3, (u + 3) % 4)
                load_e(i + 3, (u + 3) % 4)

        for j in range(3):
            load_idx(j, j)
            load_e(j, j)
        wait_idx(0, 0)
        wait_e(0, 0)
        issue_gadd(0, 0)

        HEAD = 4
        MAIN = HEAD + ((NCHUNK - 4 - HEAD) // 4) * 4
        for i in range(HEAD):
            step(i, i % 4)

        def quad(p, carry):
            i0 = HEAD + 4 * p
            for u in range(4):
                step(i0 + u, u)
            return carry

        lax.fori_loop(0, (MAIN - HEAD) // 4, quad, 0)
        for i in range(MAIN, NCHUNK):
            step(i, i % 4)
        wait_scatter((NCHUNK - 1) % 4, (NCHUNK - 1) % 4)
        plsc.subcore_barrier()

        def ocopy(i, carry):
            r0 = my_row0 + i * ZR
            pltpu.sync_copy(agg_sh.at[pl.ds(r0, ZR)],
                            out_hbm.at[c].at[pl.ds(r0, ZR)])
            return carry

        lax.fori_loop(0, NZB, ocopy, 0)

        @pl.when(s == 0)
        def _out_rem():
            pltpu.sync_copy(agg_sh.at[pl.ds(REM0, REM)],
                            out_hbm.at[c].at[pl.ds(REM0, REM)])

    return sc_layer


_SC_LAYER = _make_sc_layer()


def kernel(x, edge_index, edge_attr, W_init, b_init, W_edge, W_msg, b_msg, ln_g, ln_b):
    src = edge_index[0].astype(jnp.int32)
    dst = edge_index[1].astype(jnp.int32)
    perm = jnp.argsort(src)
    src = src[perm]
    dst = dst[perm]
    edge_attr = edge_attr[perm]

    h = pl.pallas_call(
        _mm_bias_body,
        grid=(5,),
        in_specs=[
            pl.BlockSpec((2000, D), lambda i: (i, 0)),
            pl.BlockSpec((D, D), lambda i: (0, 0)),
            pl.BlockSpec((1, D), lambda i: (0, 0)),
        ],
        out_specs=pl.BlockSpec((2000, D), lambda i: (i, 0)),
        out_shape=jax.ShapeDtypeStruct((N_NODES, D), jnp.float32),
    )(x, W_init, b_init.reshape(1, D))

    eps = [
        pl.pallas_call(
            _edge_mm_body,
            grid=(40,),
            in_specs=[
                pl.BlockSpec((8000, 16), lambda b: (b, 0)),
                pl.BlockSpec((1, 16, D), lambda b, l=l: (l, 0, 0)),
            ],
            out_specs=pl.BlockSpec((8000, D), lambda b: (b, 0)),
            out_shape=jax.ShapeDtypeStruct((N_EDGES, D), jnp.float32),
        )(edge_attr, W_edge)
        for l in range(3)
    ]

    update_body = _make_update_body()
    for l in range(3):
        agg = _SC_LAYER(h, eps[l], src, dst)
        h = pl.pallas_call(
            update_body,
            grid=(5,),
            in_specs=[
                pl.BlockSpec((2000, D), lambda i: (i, 0)),
                pl.BlockSpec((1, 2000, D), lambda i: (0, i, 0)),
                pl.BlockSpec((1, 2000, D), lambda i: (1, i, 0)),
                pl.BlockSpec((1, D, D), lambda i, l=l: (l, 0, 0)),
                pl.BlockSpec((1, 1, D), lambda i, l=l: (l, 0, 0)),
                pl.BlockSpec((1, 1, D), lambda i, l=l: (l, 0, 0)),
                pl.BlockSpec((1, 1, D), lambda i, l=l: (l, 0, 0)),
            ],
            out_specs=pl.BlockSpec((2000, D), lambda i: (i, 0)),
            out_shape=jax.ShapeDtypeStruct((N_NODES, D), jnp.float32),
        )(h, agg, agg, W_msg, b_msg.reshape(3, 1, D), ln_g.reshape(3, 1, D),
          ln_b.reshape(3, 1, D))
    return h


# speedup vs baseline: 2.1220x; 2.1220x over previous
import functools

import jax
import jax.numpy as jnp
from jax import lax
from jax.experimental import pallas as pl
from jax.experimental.pallas import tpu as pltpu
from jax.experimental.pallas import tpu_sc as plsc

N_NODES = 10000
N_EDGES = 320000
D = 128

NC = 2
NS = 16
NW = NC * NS
EPW = N_EDGES // NW
K = 80
NCHUNK = EPW // K
TPR = 624
ZR = 16
NZB = TPR // ZR
REM0 = NS * TPR
REM = N_NODES - REM0


def _mm_bias_body(x_ref, w_ref, b_ref, o_ref):
    o_ref[...] = (
        jnp.dot(x_ref[...], w_ref[...], preferred_element_type=jnp.float32)
        + b_ref[...]
    )


def _edge_mm_body(ea_ref, w_ref, o_ref):
    o_ref[...] = jnp.dot(ea_ref[...], w_ref[0],
                         preferred_element_type=jnp.float32)


def _make_update_body():
    def body(h_ref, a0_ref, a1_ref, w_ref, b_ref, g_ref, bb_ref, o_ref):
        s = h_ref[...] + a0_ref[0] + a1_ref[0]
        t = (
            jnp.dot(s, w_ref[0], preferred_element_type=jnp.float32)
            + b_ref[0]
        )
        t = jnp.maximum(t, 0.0)
        mu = jnp.mean(t, axis=-1, keepdims=True)
        var = jnp.mean((t - mu) ** 2, axis=-1, keepdims=True)
        o_ref[...] = (t - mu) * lax.rsqrt(var + 1e-5) * g_ref[0] + bb_ref[0]

    return body


def _make_sc_layer():
    mesh = plsc.VectorSubcoreMesh(core_axis_name="c", subcore_axis_name="s")

    @functools.partial(
        pl.kernel,
        out_type=jax.ShapeDtypeStruct((NC, N_NODES, D), jnp.float32),
        mesh=mesh,
        scratch_types=[
            pltpu.VMEM((4, K), jnp.int32),
            pltpu.VMEM((4, K), jnp.int32),
            pltpu.VMEM((4, K, D), jnp.float32),
            pltpu.VMEM((ZR, D), jnp.float32),
            pltpu.VMEM_SHARED((N_NODES, D), jnp.float32),
            pltpu.SemaphoreType.DMA((4,)),
            pltpu.SemaphoreType.DMA((4,)),
            pltpu.SemaphoreType.DMA((4,)),
            pltpu.SemaphoreType.DMA((4,)),
            pltpu.SemaphoreType.DMA((1,)),
        ],
    )
    def sc_layer(h_hbm, e_hbm, src_hbm, dst_hbm, out_hbm,
                 sring, dring, ebuf, zbuf, agg_sh, isem, gsem, esem, ssem,
                 zsem):
        c = lax.axis_index("c")
        s = lax.axis_index("s")
        w = c * NS + s
        base = w * EPW
        my_row0 = s * TPR

        zero = jnp.zeros((16,), jnp.float32)

        def zrow(r, carry):
            for j in range(8):
                zbuf[r, pl.ds(j * 16, 16)] = zero
            return carry

        lax.fori_loop(0, ZR, zrow, 0)

        def zcopy(i, carry):
            pltpu.sync_copy(zbuf, agg_sh.at[pl.ds(my_row0 + i * ZR, ZR)])
            return carry

        lax.fori_loop(0, NZB, zcopy, 0)

        @pl.when(s == 0)
        def _zero_rem():
            pltpu.sync_copy(zbuf.at[pl.ds(0, REM)],
                            agg_sh.at[pl.ds(REM0, REM)])

        def load_idx(i, sl):
            pltpu.async_copy(src_hbm.at[pl.ds(base + i * K, K)],
                             sring.at[sl], isem.at[sl])
            pltpu.async_copy(dst_hbm.at[pl.ds(base + i * K, K)],
                             dring.at[sl], isem.at[sl])

        def wait_idx(i, sl):
            pltpu.make_async_copy(src_hbm.at[pl.ds(base + i * K, K)],
                                  sring.at[sl], isem.at[sl]).wait()
            pltpu.make_async_copy(dst_hbm.at[pl.ds(base + i * K, K)],
                                  dring.at[sl], isem.at[sl]).wait()

        def load_e(i, b):
            pltpu.async_copy(e_hbm.at[pl.ds(base + i * K, K)],
                             ebuf.at[b], esem.at[b])

        def wait_e(i, b):
            pltpu.make_async_copy(e_hbm.at[pl.ds(base + i * K, K)],
                                  ebuf.at[b], esem.at[b]).wait()

        def issue_gadd(sl, b):
            pltpu.async_copy(h_hbm.at[sring.at[sl]], ebuf.at[b], gsem.at[b],
                             add=True)

        def wait_gadd(sl, b):
            pltpu.make_async_copy(h_hbm.at[sring.at[sl]], ebuf.at[b],
                                  gsem.at[b]).wait()

        def compute(b):
            def crow(r, inner):
                for j in range(8):
                    sl = pl.ds(j * 16, 16)
                    ebuf[b, r, sl] = jnp.maximum(ebuf[b, r, sl], 0.0)
                return inner

            lax.fori_loop(0, K, crow, 0)

        def issue_scatter(b, sl):
            pltpu.async_copy(ebuf.at[b], agg_sh.at[dring.at[sl]],
                             ssem.at[b], add=True)

        def wait_scatter(b, sl):
            pltpu.make_async_copy(ebuf.at[b], agg_sh.at[dring.at[sl]],
                                  ssem.at[b]).wait()

        def step(i, u, last=NCHUNK):
            if isinstance(i, int):
                do_prep = i + 1 < last
                do_wsc = i >= 1
                do_load = i + 3 < last
            else:
                do_prep = do_wsc = do_load = True
            if do_prep:
                wait_idx(i + 1, (u + 1) % 4)
                wait_e(i + 1, (u + 1) % 4)
                issue_gadd((u + 1) % 4, (u + 1) % 4)
            wait_gadd(u % 4, u)
            compute(u)
            issue_scatter(u, u)
            if do_wsc:
                wait_scatter((u + 3) % 4, (u + 3) % 4)
            if do_load:
                load_idx(i + 3, (u + 3) % 4)
                load_e(i + 3, (u + 3) % 4)

        for j in range(3):
            load_idx(j, j)
            load_e(j, j)
        wait_idx(0, 0)
        wait_e(0, 0)
        issue_gadd(0, 0)

        HEAD = 4
        MAIN = HEAD + ((NCHUNK - 4 - HEAD) // 4) * 4
        for i in range(HEAD):
            step(i, i % 4)

        def quad(p, carry):
            i0 = HEAD + 4 * p
            for u in range(4):
                step(i0 + u, u)
            return carry

        lax.fori_loop(0, (MAIN - HEAD) // 4, quad, 0)
        for i in range(MAIN, NCHUNK):
            step(i, i % 4)
        wait_scatter((NCHUNK - 1) % 4, (NCHUNK - 1) % 4)
        plsc.subcore_barrier()

        def ocopy(i, carry):
            r0 = my_row0 + i * ZR
            pltpu.sync_copy(agg_sh.at[pl.ds(r0, ZR)],
                            out_hbm.at[c].at[pl.ds(r0, ZR)])
            return carry

        lax.fori_loop(0, NZB, ocopy, 0)

        @pl.when(s == 0)
        def _out_rem():
            pltpu.sync_copy(agg_sh.at[pl.ds(REM0, REM)],
                            out_hbm.at[c].at[pl.ds(REM0, REM)])

    return sc_layer


_SC_LAYER = _make_sc_layer()


def kernel(x, edge_index, edge_attr, W_init, b_init, W_edge, W_msg, b_msg, ln_g, ln_b):
    src = edge_index[0].astype(jnp.int32)
    dst = edge_index[1].astype(jnp.int32)

    h = pl.pallas_call(
        _mm_bias_body,
        grid=(5,),
        in_specs=[
            pl.BlockSpec((2000, D), lambda i: (i, 0)),
            pl.BlockSpec((D, D), lambda i: (0, 0)),
            pl.BlockSpec((1, D), lambda i: (0, 0)),
        ],
        out_specs=pl.BlockSpec((2000, D), lambda i: (i, 0)),
        out_shape=jax.ShapeDtypeStruct((N_NODES, D), jnp.float32),
    )(x, W_init, b_init.reshape(1, D))

    eps = [
        pl.pallas_call(
            _edge_mm_body,
            grid=(40,),
            in_specs=[
                pl.BlockSpec((8000, 16), lambda b: (b, 0)),
                pl.BlockSpec((1, 16, D), lambda b, l=l: (l, 0, 0)),
            ],
            out_specs=pl.BlockSpec((8000, D), lambda b: (b, 0)),
            out_shape=jax.ShapeDtypeStruct((N_EDGES, D), jnp.float32),
        )(edge_attr, W_edge)
        for l in range(3)
    ]

    update_body = _make_update_body()
    for l in range(3):
        agg = _SC_LAYER(h, eps[l], src, dst)
        h = pl.pallas_call(
            update_body,
            grid=(5,),
            in_specs=[
                pl.BlockSpec((2000, D), lambda i: (i, 0)),
                pl.BlockSpec((1, 2000, D), lambda i: (0, i, 0)),
                pl.BlockSpec((1, 2000, D), lambda i: (1, i, 0)),
                pl.BlockSpec((1, D, D), lambda i, l=l: (l, 0, 0)),
                pl.BlockSpec((1, 1, D), lambda i, l=l: (l, 0, 0)),
                pl.BlockSpec((1, 1, D), lambda i, l=l: (l, 0, 0)),
                pl.BlockSpec((1, 1, D), lambda i, l=l: (l, 0, 0)),
            ],
            out_specs=pl.BlockSpec((2000, D), lambda i: (i, 0)),
            out_shape=jax.ShapeDtypeStruct((N_NODES, D), jnp.float32),
        )(h, agg, agg, W_msg, b_msg.reshape(3, 1, D), ln_g.reshape(3, 1, D),
          ln_b.reshape(3, 1, D))
    return h
